# baseline (speedup 1.0000x reference)
import functools

import jax
import jax.numpy as jnp
from jax import lax
from jax.experimental import pallas as pl
from jax.experimental.pallas import tpu as pltpu
from jax.experimental.pallas import tpu_sc as plsc

N = 10000
E = 320000
D = 128
DH = D // 2
DE = 16
NLAYER = 3

NC = 2
NS = 16
NW = NC * NS
LANES = 16

CH = 128
NCHUNK = E // CH
CHUNK_REM = NCHUNK % NW

EPT = E // NW
CHP = 64
NCHE = 156
TAILO = NCHE * CHP
TAILE = EPT - TAILO

ROW_MAIN = 624
ROW_TAIL_OFF = 16 * ROW_MAIN
ROW_TAIL = N - ROW_TAIL_OFF

_MM = dict(preferred_element_type=jnp.float32, precision=lax.Precision.DEFAULT)


def _dot(x, w):
    return jnp.dot(x.astype(jnp.bfloat16), w.astype(jnp.bfloat16), **_MM)


def _dot_wb(x, w):
    return jnp.dot(x, w.astype(jnp.bfloat16).astype(jnp.float32),
                   preferred_element_type=jnp.float32,
                   precision=lax.Precision.HIGHEST)


def _rne_bf16(x):
    u = lax.bitcast_convert_type(x, jnp.uint32)
    r = (u + jnp.uint32(0x7FFF) + ((u >> jnp.uint32(16)) & jnp.uint32(1)))
    r = r & jnp.uint32(0xFFFF0000)
    return lax.bitcast_convert_type(r, jnp.float32)


def _pack_cols(x):
    lo = lax.bitcast_convert_type(x[:, :DH].astype(jnp.bfloat16), jnp.uint16)
    hi = lax.bitcast_convert_type(x[:, DH:].astype(jnp.bfloat16), jnp.uint16)
    word = lo.astype(jnp.uint32) | (hi.astype(jnp.uint32) << 16)
    return lax.bitcast_convert_type(word, jnp.float32)


def _unpack_pair(v):
    u = lax.bitcast_convert_type(v, jnp.uint32)
    lo = lax.bitcast_convert_type(u << jnp.uint32(16), jnp.float32)
    hi = lax.bitcast_convert_type(u & jnp.uint32(0xFFFF0000), jnp.float32)
    return lo, hi


def _c_body(ea_ref, we_ref, b1_ref, out_ref):
    out_ref[0] = _pack_cols(_dot(ea_ref[...], we_ref[0]) + b1_ref[0])


def _edge_proj(edge_attr, wes, b1s, be=4000):
    grid = (NLAYER, E // be)
    return pl.pallas_call(
        _c_body,
        grid=grid,
        in_specs=[
            pl.BlockSpec((be, DE), lambda l, j: (j, 0)),
            pl.BlockSpec((1, DE, D), lambda l, j: (l, 0, 0)),
            pl.BlockSpec((1, 1, D), lambda l, j: (l, 0, 0)),
        ],
        out_specs=pl.BlockSpec((1, be, DH), lambda l, j: (l, j, 0)),
        out_shape=jax.ShapeDtypeStruct((NLAYER, E, DH), jnp.float32),
    )(edge_attr, wes, b1s)


def _ab_body(h_ref, wsrc_ref, wdst_ref, a_ref, b_ref):
    hblk = h_ref[...]
    a_ref[...] = _dot(hblk, wsrc_ref[0])
    b_ref[...] = _dot(hblk, wdst_ref[0])


def _ab0(h, wsrcs, wdsts, bn=1000):
    return pl.pallas_call(
        _ab_body,
        grid=(N // bn,),
        in_specs=[
            pl.BlockSpec((bn, D), lambda i: (i, 0)),
            pl.BlockSpec((1, D, D), lambda i: (0, 0, 0)),
            pl.BlockSpec((1, D, D), lambda i: (0, 0, 0)),
        ],
        out_specs=[
            pl.BlockSpec((bn, D), lambda i: (i, 0)),
            pl.BlockSpec((bn, D), lambda i: (i, 0)),
        ],
        out_shape=[
            jax.ShapeDtypeStruct((N, D), jnp.float32),
            jax.ShapeDtypeStruct((N, D), jnp.float32),
        ],
    )(h, wsrcs, wdsts)


def _update_body(emit_ab, h_ref, s0_ref, s1_ref, d0_ref, d1_ref,
                 w2_ref, b2_ref, g1h_ref, g1a_ref, c1_ref, g2_ref, c2_ref,
                 *rest):
    if emit_ab:
        wsrc_ref, wdst_ref, outh_ref, outa_ref, outb_ref = rest
    else:
        (outh_ref,) = rest
    s = s0_ref[...] + s1_ref[...]
    deg = (d0_ref[:, 0] + d1_ref[:, 0])[:, None]
    agg = _dot_wb(s, w2_ref[0]) + deg * b2_ref[0]
    hblk = h_ref[...]
    u = _dot(hblk, g1h_ref[0]) + _dot(agg, g1a_ref[0])
    u = jnp.maximum(u + c1_ref[0], 0.0)
    hn = hblk + _dot(u, g2_ref[0]) + c2_ref[0]
    outh_ref[...] = hn
    if emit_ab:
        outa_ref[...] = _dot(hn, wsrc_ref[0])
        outb_ref[...] = _dot(hn, wdst_ref[0])


def _update(layer, h, s_parts, d0, d1, wts, bn=1000):
    emit_ab = layer + 1 < NLAYER
    wblk = lambda l: pl.BlockSpec((1, D, D), lambda i, _l=l: (_l, 0, 0))
    vblk = lambda l: pl.BlockSpec((1, 1, D), lambda i, _l=l: (_l, 0, 0))
    nblk = pl.BlockSpec((bn, D), lambda i: (i, 0))
    dblk = pl.BlockSpec((bn, LANES), lambda i: (i, 0))
    in_specs = [nblk, nblk, nblk, dblk, dblk,
                wblk(layer), vblk(layer), wblk(layer), wblk(layer),
                vblk(layer), wblk(layer), vblk(layer)]
    args = [h, *s_parts, d0, d1,
            wts["w2"], wts["b2"], wts["g1h"], wts["g1a"],
            wts["c1"], wts["g2"], wts["c2"]]
    out_specs = [nblk]
    out_shape = [jax.ShapeDtypeStruct((N, D), jnp.float32)]
    if emit_ab:
        in_specs += [wblk(layer + 1), wblk(layer + 1)]
        args += [wts["wsrc"], wts["wdst"]]
        out_specs += [nblk, nblk]
        out_shape += [jax.ShapeDtypeStruct((N, D), jnp.float32),
                      jax.ShapeDtypeStruct((N, D), jnp.float32)]
    return pl.pallas_call(
        functools.partial(_update_body, emit_ab),
        grid=(N // bn,),
        in_specs=in_specs,
        out_specs=out_specs,
        out_shape=out_shape,
    )(*args)


_MESH = plsc.VectorSubcoreMesh(core_axis_name="c", subcore_axis_name="s")


def _zero_rows(buf, nrows, width):
    def row(r, _):
        for j in range(width // LANES):
            buf[r, pl.ds(j * LANES, LANES)] = jnp.zeros((LANES,), jnp.float32)
        return 0
    lax.fori_loop(0, nrows, row, 0)


def _worker_chunks(wid):
    return jnp.where(wid < CHUNK_REM, NCHUNK // NW + 1, NCHUNK // NW)


def _zero_shared_slice(sid, zbuf, shared):
    base = sid * ROW_MAIN
    for off, sz in ((0, 96), (96, 96), (192, 96), (288, 96),
                    (384, 96), (480, 96), (576, 48)):
        pltpu.sync_copy(zbuf.at[pl.ds(0, sz)], shared.at[pl.ds(base + off, sz)])

    @pl.when(sid == NS - 1)
    def _():
        pltpu.sync_copy(zbuf.at[pl.ds(0, ROW_TAIL)],
                        shared.at[pl.ds(ROW_TAIL_OFF, ROW_TAIL)])


def _dump_shared_slice(cid, sid, shared, out0, out1):
    base = sid * ROW_MAIN

    @pl.when(cid == 0)
    def _():
        pltpu.sync_copy(shared.at[pl.ds(base, ROW_MAIN)],
                        out0.at[pl.ds(base, ROW_MAIN)])

    @pl.when(cid == 1)
    def _():
        pltpu.sync_copy(shared.at[pl.ds(base, ROW_MAIN)],
                        out1.at[pl.ds(base, ROW_MAIN)])

    @pl.when((cid == 0) & (sid == NS - 1))
    def _():
        pltpu.sync_copy(shared.at[pl.ds(ROW_TAIL_OFF, ROW_TAIL)],
                        out0.at[pl.ds(ROW_TAIL_OFF, ROW_TAIL)])

    @pl.when((cid == 1) & (sid == NS - 1))
    def _():
        pltpu.sync_copy(shared.at[pl.ds(ROW_TAIL_OFF, ROW_TAIL)],
                        out1.at[pl.ds(ROW_TAIL_OFF, ROW_TAIL)])


@functools.partial(
    pl.kernel,
    out_type=(jax.ShapeDtypeStruct((N, LANES), jnp.float32),
              jax.ShapeDtypeStruct((N, LANES), jnp.float32)),
    mesh=_MESH,
    scratch_types=[
        pltpu.VMEM((CH,), jnp.int32),
        pltpu.VMEM((CH, LANES), jnp.float32),
        pltpu.MemorySpace.VMEM_SHARED((N, LANES), jnp.float32),
    ],
)
def _deg_kernel(dst_hbm, out0, out1, idx_d, ones_v, deg_shared):
    cid = lax.axis_index("c")
    sid = lax.axis_index("s")
    wid = sid * NC + cid
    _zero_rows(ones_v, CH, LANES)
    _zero_shared_slice(sid, ones_v, deg_shared)
    plsc.subcore_barrier()

    def fill(r, _):
        ones_v[r, pl.ds(0, LANES)] = jnp.full((LANES,), 1.0, jnp.float32)
        return 0
    lax.fori_loop(0, CH, fill, 0)

    def body(i, _):
        base = (wid + NW * i) * CH
        pltpu.sync_copy(dst_hbm.at[pl.ds(base, CH)], idx_d)
        pltpu.sync_copy(ones_v, deg_shared.at[idx_d], add=True)
        return 0
    lax.fori_loop(0, _worker_chunks(wid), body, 0)
    plsc.subcore_barrier()
    _dump_shared_slice(cid, sid, deg_shared, out0, out1)


def _make_edge_kernel(layer):
    @functools.partial(
        pl.kernel,
        out_type=(jax.ShapeDtypeStruct((N, D), jnp.float32),
                  jax.ShapeDtypeStruct((N, D), jnp.float32)),
        mesh=_MESH,
        scratch_types=[
            pltpu.VMEM((CHP,), jnp.int32),
            pltpu.VMEM((CHP,), jnp.int32),
            pltpu.VMEM((CHP,), jnp.int32),
            pltpu.VMEM((CHP,), jnp.int32),
            pltpu.VMEM((CHP,), jnp.int32),
            pltpu.VMEM((CHP,), jnp.int32),
            pltpu.VMEM((CHP,), jnp.int32),
            pltpu.VMEM((CHP,), jnp.int32),
            pltpu.VMEM((TAILE,), jnp.int32),
            pltpu.VMEM((TAILE,), jnp.int32),
            pltpu.VMEM((CHP, D), jnp.float32),
            pltpu.VMEM((CHP, D), jnp.float32),
            pltpu.VMEM((CHP, DH), jnp.float32),
            pltpu.VMEM((CHP, D), jnp.float32),
            pltpu.VMEM((CHP, D), jnp.float32),
            pltpu.VMEM((CHP, DH), jnp.float32),
            pltpu.MemorySpace.VMEM_SHARED((N, D), jnp.float32),
            pltpu.SemaphoreType.DMA,
            pltpu.SemaphoreType.DMA,
            pltpu.SemaphoreType.DMA,
            pltpu.SemaphoreType.DMA,
            pltpu.SemaphoreType.DMA,
            pltpu.SemaphoreType.DMA,
        ],
    )
    def _edge_kernel(a_hbm, b_hbm, c_hbm, src_hbm, dst_hbm, out0, out1,
                     i0s, i0d, i1s, i1d, i2s, i2d, i3s, i3d, ts, td,
                     a0, b0, c0, a1, b1, c1, s_shared,
                     sem0, sem1, is0, is1, is2, is3):
        cid = lax.axis_index("c")
        sid = lax.axis_index("s")
        wid = sid * NC + cid
        base_e = wid * EPT
        sets = ((a0, b0, c0, sem0), (a1, b1, c1, sem1))
        slots = ((i0s, i0d, is0), (i1s, i1d, is1),
                 (i2s, i2d, is2), (i3s, i3d, is3))

        _zero_rows(a0, CHP, D)
        _zero_shared_slice(sid, a0, s_shared)
        plsc.subcore_barrier()

        def issue_idx(c, q):
            islot_s, islot_d, isem = slots[q]
            base = base_e + c * CHP
            pltpu.async_copy(src_hbm.at[pl.ds(base, CHP)], islot_s, isem)
            pltpu.async_copy(dst_hbm.at[pl.ds(base, CHP)], islot_d, isem)

        def wait_idx(q):
            islot_s, islot_d, isem = slots[q]
            pltpu.make_async_copy(
                src_hbm.at[pl.ds(0, CHP)], islot_s, isem).wait()
            pltpu.make_async_copy(
                dst_hbm.at[pl.ds(0, CHP)], islot_d, isem).wait()

        def issue_gathers(c, q):
            islot_s, islot_d, _ = slots[q]
            ab, bb, cb, sem = sets[q % 2]
            wait_idx(q)
            pltpu.async_copy(a_hbm.at[islot_s], ab, sem)
            pltpu.async_copy(b_hbm.at[islot_d], bb, sem)
            pltpu.async_copy(
                c_hbm.at[layer, pl.ds(base_e + c * CHP, CHP)], cb, sem)

        def wait_rows(q):
            islot_s, islot_d, _ = slots[q]
            ab, bb, cb, sem = sets[q % 2]
            pltpu.make_async_copy(a_hbm.at[islot_s], ab, sem).wait()
            pltpu.make_async_copy(b_hbm.at[islot_d], bb, sem).wait()
            pltpu.make_async_copy(
                c_hbm.at[layer, pl.ds(base_e, CHP)], cb, sem).wait()

        def compute_scatter(q):
            islot_s, islot_d, _ = slots[q]
            ab, bb, cb, _sem = sets[q % 2]

            def crow(r, _):
                for g in range(DH // LANES):
                    sl = pl.ds(g * LANES, LANES)
                    sh = pl.ds(DH + g * LANES, LANES)
                    lc, hc = _unpack_pair(cb[r, sl])
                    ab[r, sl] = _rne_bf16(
                        jnp.maximum(ab[r, sl] + bb[r, sl] + lc, 0.0))
                    ab[r, sh] = _rne_bf16(
                        jnp.maximum(ab[r, sh] + bb[r, sh] + hc, 0.0))
                return 0
            lax.fori_loop(0, CHP, crow, 0)
            pltpu.sync_copy(ab, s_shared.at[islot_d], add=True)

        for c in range(4):
            issue_idx(c, c)
        issue_gathers(0, 0)
        for c in range(1, 4):
            issue_gathers(c, c)
            wait_rows(c - 1)
            compute_scatter(c - 1)
            issue_idx(c + 3, (c + 3) % 4)

        def body(k, _):
            cb4 = 4 * k
            for q in range(4):
                c = cb4 + q
                issue_gathers(c, q)
                wait_rows((q + 3) % 4)
                compute_scatter((q + 3) % 4)

                @pl.when(c + 3 < NCHE)
                def _():
                    issue_idx(c + 3, (q + 3) % 4)
            return 0
        lax.fori_loop(1, NCHE // 4, body, 0)

        wait_rows(3)
        compute_scatter(3)

        pltpu.sync_copy(src_hbm.at[pl.ds(base_e + TAILO, TAILE)], ts)
        pltpu.sync_copy(dst_hbm.at[pl.ds(base_e + TAILO, TAILE)], td)
        pltpu.async_copy(a_hbm.at[ts], a0.at[pl.ds(0, TAILE)], sem0)
        pltpu.async_copy(b_hbm.at[td], b0.at[pl.ds(0, TAILE)], sem0)
        pltpu.async_copy(c_hbm.at[layer, pl.ds(base_e + TAILO, TAILE)],
                         c0.at[pl.ds(0, TAILE)], sem0)
        pltpu.make_async_copy(a_hbm.at[ts], a0.at[pl.ds(0, TAILE)],
                              sem0).wait()
        pltpu.make_async_copy(b_hbm.at[td], b0.at[pl.ds(0, TAILE)],
                              sem0).wait()
        pltpu.make_async_copy(c_hbm.at[layer, pl.ds(base_e + TAILO, TAILE)],
                              c0.at[pl.ds(0, TAILE)], sem0).wait()

        def trow(r, _):
            for g in range(DH // LANES):
                sl = pl.ds(g * LANES, LANES)
                sh = pl.ds(DH + g * LANES, LANES)
                lc, hc = _unpack_pair(c0[r, sl])
                a0[r, sl] = _rne_bf16(
                    jnp.maximum(a0[r, sl] + b0[r, sl] + lc, 0.0))
                a0[r, sh] = _rne_bf16(
                    jnp.maximum(a0[r, sh] + b0[r, sh] + hc, 0.0))
            return 0
        lax.fori_loop(0, TAILE, trow, 0)
        pltpu.sync_copy(a0.at[pl.ds(0, TAILE)], s_shared.at[td], add=True)

        plsc.subcore_barrier()
        _dump_shared_slice(cid, sid, s_shared, out0, out1)

    return _edge_kernel


_EDGE_KERNELS = [_make_edge_kernel(l) for l in range(NLAYER)]


def kernel(h, edge_attr, edge_index, params):
    src = edge_index[0]
    dst = edge_index[1]

    wts = {
        "wsrc": jnp.stack([p["phi"][0][0][:D] for p in params]),
        "wdst": jnp.stack([p["phi"][0][0][D:2 * D] for p in params]),
        "we": jnp.stack([p["phi"][0][0][2 * D:] for p in params]),
        "b1": jnp.stack([p["phi"][0][1][None, :] for p in params]),
        "w2": jnp.stack([p["phi"][1][0] for p in params]),
        "b2": jnp.stack([p["phi"][1][1][None, :] for p in params]),
        "g1h": jnp.stack([p["gamma"][0][0][:D] for p in params]),
        "g1a": jnp.stack([p["gamma"][0][0][D:] for p in params]),
        "c1": jnp.stack([p["gamma"][0][1][None, :] for p in params]),
        "g2": jnp.stack([p["gamma"][1][0] for p in params]),
        "c2": jnp.stack([p["gamma"][1][1][None, :] for p in params]),
    }

    c_all = _edge_proj(edge_attr, wts["we"], wts["b1"])
    a, b = _ab0(h, wts["wsrc"], wts["wdst"])
    d0, d1 = _deg_kernel(dst)

    for l in range(NLAYER):
        s_parts = _EDGE_KERNELS[l](a, b, c_all, src, dst)
        outs = _update(l, h, s_parts, d0, d1, wts)
        if l + 1 < NLAYER:
            h, a, b = outs
        else:
            (h,) = outs
    return h


# speedup vs baseline: 1.0215x; 1.0215x over previous
import functools

import jax
import jax.numpy as jnp
from jax import lax
from jax.experimental import pallas as pl
from jax.experimental.pallas import tpu as pltpu
from jax.experimental.pallas import tpu_sc as plsc

N = 10000
E = 320000
D = 128
DH = D // 2
DE = 16
NLAYER = 3

NC = 2
NS = 16
NW = NC * NS
LANES = 16

CH = 128
NCHUNK = E // CH
CHUNK_REM = NCHUNK % NW

EPT = E // NW
CHP = 64
NCHE = 156
TAILO = NCHE * CHP
TAILE = EPT - TAILO

ROW_MAIN = 624
ROW_TAIL_OFF = 16 * ROW_MAIN
ROW_TAIL = N - ROW_TAIL_OFF

_MM = dict(preferred_element_type=jnp.float32, precision=lax.Precision.DEFAULT)


def _dot(x, w):
    return jnp.dot(x.astype(jnp.bfloat16), w.astype(jnp.bfloat16), **_MM)


def _dot_wb(x, w):
    return jnp.dot(x, w.astype(jnp.bfloat16).astype(jnp.float32),
                   preferred_element_type=jnp.float32,
                   precision=lax.Precision.HIGHEST)


def _rne_bf16(x):
    u = lax.bitcast_convert_type(x, jnp.uint32)
    r = (u + jnp.uint32(0x7FFF) + ((u >> jnp.uint32(16)) & jnp.uint32(1)))
    r = r & jnp.uint32(0xFFFF0000)
    return lax.bitcast_convert_type(r, jnp.float32)


def _c_body(ea_ref, we_ref, b1_ref, out_ref):
    out_ref[0] = _dot(ea_ref[...], we_ref[0]) + b1_ref[0]


def _edge_proj(edge_attr, wes, b1s, be=4000):
    grid = (NLAYER, E // be)
    return pl.pallas_call(
        _c_body,
        grid=grid,
        in_specs=[
            pl.BlockSpec((be, DE), lambda l, j: (j, 0)),
            pl.BlockSpec((1, DE, D), lambda l, j: (l, 0, 0)),
            pl.BlockSpec((1, 1, D), lambda l, j: (l, 0, 0)),
        ],
        out_specs=pl.BlockSpec((1, be, D), lambda l, j: (l, j, 0)),
        out_shape=jax.ShapeDtypeStruct((NLAYER, E, D), jnp.float32),
    )(edge_attr, wes, b1s)


def _ab_body(h_ref, wsrc_ref, wdst_ref, a_ref, b_ref):
    hblk = h_ref[...]
    a_ref[...] = _dot(hblk, wsrc_ref[0])
    b_ref[...] = _dot(hblk, wdst_ref[0])


def _ab0(h, wsrcs, wdsts, bn=1000):
    return pl.pallas_call(
        _ab_body,
        grid=(N // bn,),
        in_specs=[
            pl.BlockSpec((bn, D), lambda i: (i, 0)),
            pl.BlockSpec((1, D, D), lambda i: (0, 0, 0)),
            pl.BlockSpec((1, D, D), lambda i: (0, 0, 0)),
        ],
        out_specs=[
            pl.BlockSpec((bn, D), lambda i: (i, 0)),
            pl.BlockSpec((bn, D), lambda i: (i, 0)),
        ],
        out_shape=[
            jax.ShapeDtypeStruct((N, D), jnp.float32),
            jax.ShapeDtypeStruct((N, D), jnp.float32),
        ],
    )(h, wsrcs, wdsts)


def _update_body(emit_ab, h_ref, s0_ref, s1_ref, d0_ref, d1_ref,
                 w2_ref, b2_ref, g1h_ref, g1a_ref, c1_ref, g2_ref, c2_ref,
                 *rest):
    if emit_ab:
        wsrc_ref, wdst_ref, outh_ref, outa_ref, outb_ref = rest
    else:
        (outh_ref,) = rest
    s = s0_ref[...] + s1_ref[...]
    deg = (d0_ref[:, 0] + d1_ref[:, 0])[:, None]
    agg = _dot_wb(s, w2_ref[0]) + deg * b2_ref[0]
    hblk = h_ref[...]
    u = _dot(hblk, g1h_ref[0]) + _dot(agg, g1a_ref[0])
    u = jnp.maximum(u + c1_ref[0], 0.0)
    hn = hblk + _dot(u, g2_ref[0]) + c2_ref[0]
    outh_ref[...] = hn
    if emit_ab:
        outa_ref[...] = _dot(hn, wsrc_ref[0])
        outb_ref[...] = _dot(hn, wdst_ref[0])


def _update(layer, h, s_parts, d0, d1, wts, bn=1000):
    emit_ab = layer + 1 < NLAYER
    wblk = lambda l: pl.BlockSpec((1, D, D), lambda i, _l=l: (_l, 0, 0))
    vblk = lambda l: pl.BlockSpec((1, 1, D), lambda i, _l=l: (_l, 0, 0))
    nblk = pl.BlockSpec((bn, D), lambda i: (i, 0))
    dblk = pl.BlockSpec((bn, LANES), lambda i: (i, 0))
    in_specs = [nblk, nblk, nblk, dblk, dblk,
                wblk(layer), vblk(layer), wblk(layer), wblk(layer),
                vblk(layer), wblk(layer), vblk(layer)]
    args = [h, *s_parts, d0, d1,
            wts["w2"], wts["b2"], wts["g1h"], wts["g1a"],
            wts["c1"], wts["g2"], wts["c2"]]
    out_specs = [nblk]
    out_shape = [jax.ShapeDtypeStruct((N, D), jnp.float32)]
    if emit_ab:
        in_specs += [wblk(layer + 1), wblk(layer + 1)]
        args += [wts["wsrc"], wts["wdst"]]
        out_specs += [nblk, nblk]
        out_shape += [jax.ShapeDtypeStruct((N, D), jnp.float32),
                      jax.ShapeDtypeStruct((N, D), jnp.float32)]
    return pl.pallas_call(
        functools.partial(_update_body, emit_ab),
        grid=(N // bn,),
        in_specs=in_specs,
        out_specs=out_specs,
        out_shape=out_shape,
    )(*args)


_MESH = plsc.VectorSubcoreMesh(core_axis_name="c", subcore_axis_name="s")


def _zero_rows(buf, nrows, width):
    def row(r, _):
        for j in range(width // LANES):
            buf[r, pl.ds(j * LANES, LANES)] = jnp.zeros((LANES,), jnp.float32)
        return 0
    lax.fori_loop(0, nrows, row, 0)


def _worker_chunks(wid):
    return jnp.where(wid < CHUNK_REM, NCHUNK // NW + 1, NCHUNK // NW)


def _zero_shared_slice(sid, zbuf, shared):
    base = sid * ROW_MAIN
    for off, sz in ((0, 96), (96, 96), (192, 96), (288, 96),
                    (384, 96), (480, 96), (576, 48)):
        pltpu.sync_copy(zbuf.at[pl.ds(0, sz)], shared.at[pl.ds(base + off, sz)])

    @pl.when(sid == NS - 1)
    def _():
        pltpu.sync_copy(zbuf.at[pl.ds(0, ROW_TAIL)],
                        shared.at[pl.ds(ROW_TAIL_OFF, ROW_TAIL)])


def _dump_shared_slice(cid, sid, shared, out0, out1):
    base = sid * ROW_MAIN

    @pl.when(cid == 0)
    def _():
        pltpu.sync_copy(shared.at[pl.ds(base, ROW_MAIN)],
                        out0.at[pl.ds(base, ROW_MAIN)])

    @pl.when(cid == 1)
    def _():
        pltpu.sync_copy(shared.at[pl.ds(base, ROW_MAIN)],
                        out1.at[pl.ds(base, ROW_MAIN)])

    @pl.when((cid == 0) & (sid == NS - 1))
    def _():
        pltpu.sync_copy(shared.at[pl.ds(ROW_TAIL_OFF, ROW_TAIL)],
                        out0.at[pl.ds(ROW_TAIL_OFF, ROW_TAIL)])

    @pl.when((cid == 1) & (sid == NS - 1))
    def _():
        pltpu.sync_copy(shared.at[pl.ds(ROW_TAIL_OFF, ROW_TAIL)],
                        out1.at[pl.ds(ROW_TAIL_OFF, ROW_TAIL)])


@functools.partial(
    pl.kernel,
    out_type=(jax.ShapeDtypeStruct((N, LANES), jnp.float32),
              jax.ShapeDtypeStruct((N, LANES), jnp.float32)),
    mesh=_MESH,
    scratch_types=[
        pltpu.VMEM((CH,), jnp.int32),
        pltpu.VMEM((CH, LANES), jnp.float32),
        pltpu.MemorySpace.VMEM_SHARED((N, LANES), jnp.float32),
    ],
)
def _deg_kernel(dst_hbm, out0, out1, idx_d, ones_v, deg_shared):
    cid = lax.axis_index("c")
    sid = lax.axis_index("s")
    wid = sid * NC + cid
    _zero_rows(ones_v, CH, LANES)
    _zero_shared_slice(sid, ones_v, deg_shared)
    plsc.subcore_barrier()

    def fill(r, _):
        ones_v[r, pl.ds(0, LANES)] = jnp.full((LANES,), 1.0, jnp.float32)
        return 0
    lax.fori_loop(0, CH, fill, 0)

    def body(i, _):
        base = (wid + NW * i) * CH
        pltpu.sync_copy(dst_hbm.at[pl.ds(base, CH)], idx_d)
        pltpu.sync_copy(ones_v, deg_shared.at[idx_d], add=True)
        return 0
    lax.fori_loop(0, _worker_chunks(wid), body, 0)
    plsc.subcore_barrier()
    _dump_shared_slice(cid, sid, deg_shared, out0, out1)


def _make_edge_kernel(layer):
    @functools.partial(
        pl.kernel,
        out_type=(jax.ShapeDtypeStruct((N, D), jnp.float32),
                  jax.ShapeDtypeStruct((N, D), jnp.float32)),
        mesh=_MESH,
        scratch_types=[
            pltpu.VMEM((CHP,), jnp.int32),
            pltpu.VMEM((CHP,), jnp.int32),
            pltpu.VMEM((CHP,), jnp.int32),
            pltpu.VMEM((CHP,), jnp.int32),
            pltpu.VMEM((CHP,), jnp.int32),
            pltpu.VMEM((CHP,), jnp.int32),
            pltpu.VMEM((CHP,), jnp.int32),
            pltpu.VMEM((CHP,), jnp.int32),
            pltpu.VMEM((TAILE,), jnp.int32),
            pltpu.VMEM((TAILE,), jnp.int32),
            pltpu.VMEM((CHP, D), jnp.float32),
            pltpu.VMEM((CHP, D), jnp.float32),
            pltpu.VMEM((CHP, D), jnp.float32),
            pltpu.VMEM((CHP, D), jnp.float32),
            pltpu.VMEM((CHP, D), jnp.float32),
            pltpu.VMEM((CHP, D), jnp.float32),
            pltpu.MemorySpace.VMEM_SHARED((N, D), jnp.float32),
            pltpu.SemaphoreType.DMA,
            pltpu.SemaphoreType.DMA,
            pltpu.SemaphoreType.DMA,
            pltpu.SemaphoreType.DMA,
            pltpu.SemaphoreType.DMA,
            pltpu.SemaphoreType.DMA,
            pltpu.SemaphoreType.DMA,
            pltpu.SemaphoreType.DMA,
        ],
    )
    def _edge_kernel(a_hbm, b_hbm, c_hbm, src_hbm, dst_hbm, out0, out1,
                     i0s, i0d, i1s, i1d, i2s, i2d, i3s, i3d, ts, td,
                     a0, b0, c0, a1, b1, c1, s_shared,
                     sem0, sem1, ssem0, ssem1, is0, is1, is2, is3):
        cid = lax.axis_index("c")
        sid = lax.axis_index("s")
        wid = sid * NC + cid
        base_e = wid * EPT
        sets = ((a0, b0, c0, sem0, ssem0), (a1, b1, c1, sem1, ssem1))
        slots = ((i0s, i0d, is0), (i1s, i1d, is1),
                 (i2s, i2d, is2), (i3s, i3d, is3))

        _zero_rows(a0, CHP, D)
        _zero_shared_slice(sid, a0, s_shared)
        plsc.subcore_barrier()

        def issue_idx(c, q):
            islot_s, islot_d, isem = slots[q]
            base = base_e + c * CHP
            pltpu.async_copy(src_hbm.at[pl.ds(base, CHP)], islot_s, isem)
            pltpu.async_copy(dst_hbm.at[pl.ds(base, CHP)], islot_d, isem)

        def wait_idx(q):
            islot_s, islot_d, isem = slots[q]
            pltpu.make_async_copy(
                src_hbm.at[pl.ds(0, CHP)], islot_s, isem).wait()
            pltpu.make_async_copy(
                dst_hbm.at[pl.ds(0, CHP)], islot_d, isem).wait()

        def wait_scatter(setidx):
            ab = sets[setidx][0]
            ssem = sets[setidx][4]
            pltpu.make_async_copy(ab, s_shared.at[slots[0][1]], ssem).wait()

        def issue_gathers(c, q, drain_scatter=True):
            islot_s, islot_d, _ = slots[q]
            ab, bb, cb, sem, _ssem = sets[q % 2]
            if drain_scatter:
                wait_scatter(q % 2)
            wait_idx(q)
            pltpu.async_copy(a_hbm.at[islot_s], ab, sem)
            pltpu.async_copy(b_hbm.at[islot_d], bb, sem)
            pltpu.async_copy(
                c_hbm.at[layer, pl.ds(base_e + c * CHP, CHP)], cb, sem)

        def wait_rows(q):
            islot_s, islot_d, _ = slots[q]
            ab, bb, cb, sem, _ssem = sets[q % 2]
            pltpu.make_async_copy(a_hbm.at[islot_s], ab, sem).wait()
            pltpu.make_async_copy(b_hbm.at[islot_d], bb, sem).wait()
            pltpu.make_async_copy(
                c_hbm.at[layer, pl.ds(base_e, CHP)], cb, sem).wait()

        def compute_scatter(q):
            islot_s, islot_d, _ = slots[q]
            ab, bb, cb, _sem, ssem = sets[q % 2]

            def crow(r, _):
                for g in range(D // LANES):
                    sl = pl.ds(g * LANES, LANES)
                    z = jnp.maximum(ab[r, sl] + bb[r, sl] + cb[r, sl], 0.0)
                    ab[r, sl] = _rne_bf16(z)
                return 0
            lax.fori_loop(0, CHP, crow, 0)
            pltpu.async_copy(ab, s_shared.at[islot_d], ssem, add=True)

        for c in range(4):
            issue_idx(c, c)
        issue_gathers(0, 0, drain_scatter=False)
        for c in range(1, 4):
            issue_gathers(c, c, drain_scatter=(c >= 2))
            wait_rows(c - 1)
            compute_scatter(c - 1)
            issue_idx(c + 3, (c + 3) % 4)

        def body(k, _):
            cb4 = 4 * k
            for q in range(4):
                c = cb4 + q
                issue_gathers(c, q)
                wait_rows((q + 3) % 4)
                compute_scatter((q + 3) % 4)

                @pl.when(c + 3 < NCHE)
                def _():
                    issue_idx(c + 3, (q + 3) % 4)
            return 0
        lax.fori_loop(1, NCHE // 4, body, 0)

        wait_rows(3)
        compute_scatter(3)
        wait_scatter(0)

        pltpu.sync_copy(src_hbm.at[pl.ds(base_e + TAILO, TAILE)], ts)
        pltpu.sync_copy(dst_hbm.at[pl.ds(base_e + TAILO, TAILE)], td)
        pltpu.async_copy(a_hbm.at[ts], a0.at[pl.ds(0, TAILE)], sem0)
        pltpu.async_copy(b_hbm.at[td], b0.at[pl.ds(0, TAILE)], sem0)
        pltpu.async_copy(c_hbm.at[layer, pl.ds(base_e + TAILO, TAILE)],
                         c0.at[pl.ds(0, TAILE)], sem0)
        pltpu.make_async_copy(a_hbm.at[ts], a0.at[pl.ds(0, TAILE)],
                              sem0).wait()
        pltpu.make_async_copy(b_hbm.at[td], b0.at[pl.ds(0, TAILE)],
                              sem0).wait()
        pltpu.make_async_copy(c_hbm.at[layer, pl.ds(base_e + TAILO, TAILE)],
                              c0.at[pl.ds(0, TAILE)], sem0).wait()

        def trow(r, _):
            for g in range(D // LANES):
                sl = pl.ds(g * LANES, LANES)
                z = jnp.maximum(a0[r, sl] + b0[r, sl] + c0[r, sl], 0.0)
                a0[r, sl] = _rne_bf16(z)
            return 0
        lax.fori_loop(0, TAILE, trow, 0)
        pltpu.sync_copy(a0.at[pl.ds(0, TAILE)], s_shared.at[td], add=True)
        wait_scatter(1)

        plsc.subcore_barrier()
        _dump_shared_slice(cid, sid, s_shared, out0, out1)

    return _edge_kernel


_EDGE_KERNELS = [_make_edge_kernel(l) for l in range(NLAYER)]


def kernel(h, edge_attr, edge_index, params):
    src = edge_index[0]
    dst = edge_index[1]

    wts = {
        "wsrc": jnp.stack([p["phi"][0][0][:D] for p in params]),
        "wdst": jnp.stack([p["phi"][0][0][D:2 * D] for p in params]),
        "we": jnp.stack([p["phi"][0][0][2 * D:] for p in params]),
        "b1": jnp.stack([p["phi"][0][1][None, :] for p in params]),
        "w2": jnp.stack([p["phi"][1][0] for p in params]),
        "b2": jnp.stack([p["phi"][1][1][None, :] for p in params]),
        "g1h": jnp.stack([p["gamma"][0][0][:D] for p in params]),
        "g1a": jnp.stack([p["gamma"][0][0][D:] for p in params]),
        "c1": jnp.stack([p["gamma"][0][1][None, :] for p in params]),
        "g2": jnp.stack([p["gamma"][1][0] for p in params]),
        "c2": jnp.stack([p["gamma"][1][1][None, :] for p in params]),
    }

    c_all = _edge_proj(edge_attr, wts["we"], wts["b1"])
    a, b = _ab0(h, wts["wsrc"], wts["wdst"])
    d0, d1 = _deg_kernel(dst)

    for l in range(NLAYER):
        s_parts = _EDGE_KERNELS[l](a, b, c_all, src, dst)
        outs = _update(l, h, s_parts, d0, d1, wts)
        if l + 1 < NLAYER:
            h, a, b = outs
        else:
            (h,) = outs
    return h
